# Initial kernel scaffold; baseline (speedup 1.0000x reference)
import jax
import jax.numpy as jnp
from jax.experimental import pallas as pl


def kernel(x, table):
    raise NotImplementedError("write your pallas kernel here")


# speedup vs baseline: 1.0947x; 1.0947x over previous
import functools

import jax
import jax.numpy as jnp
from jax import lax
from jax.experimental import pallas as pl
from jax.experimental.pallas import tpu as pltpu
from jax.experimental.pallas import tpu_sc as plsc


@functools.partial(jax.jit, static_argnames=())
def kernel(x, table):
    B = x.shape[0] * x.shape[1]
    D = table.shape[1]
    idx = x.reshape(B).astype(jnp.int32)

    info = plsc.get_sparse_core_info()
    NW = info.num_cores * info.num_subcores
    b_per_w = B // NW
    CH = 1024
    n_ch = b_per_w // CH
    mesh = plsc.VectorSubcoreMesh(core_axis_name="c", subcore_axis_name="s")

    @functools.partial(
        pl.kernel,
        mesh=mesh,
        out_type=jax.ShapeDtypeStruct((B, D), jnp.float32),
        scratch_types=[
            pltpu.VMEM((CH,), jnp.int32),
            pltpu.VMEM((CH, D), jnp.float32),
            pltpu.SemaphoreType.DMA,
        ],
        compiler_params=pltpu.CompilerParams(use_tc_tiling_on_sc=False),
    )
    def emb(idx_hbm, table_hbm, out_hbm, idx_v, rows_v, sem):
        wid = lax.axis_index("s") * info.num_cores + lax.axis_index("c")
        base0 = wid * b_per_w

        def body(j, carry):
            base = base0 + j * CH
            pltpu.sync_copy(idx_hbm.at[pl.ds(base, CH)], idx_v)
            pltpu.async_copy(table_hbm.at[idx_v], rows_v, sem).wait()
            pltpu.sync_copy(rows_v, out_hbm.at[pl.ds(base, CH)])
            return carry

        lax.fori_loop(0, n_ch, body, 0)

    out = emb(idx, table)
    return out.reshape(x.shape[0], x.shape[1], D)


# speedup vs baseline: 1.1095x; 1.0136x over previous
import functools

import jax
import jax.numpy as jnp
from jax import lax
from jax.experimental import pallas as pl
from jax.experimental.pallas import tpu as pltpu
from jax.experimental.pallas import tpu_sc as plsc


def kernel(x, table):
    B = x.shape[0] * x.shape[1]
    D = table.shape[1]
    idx = x.reshape(B).astype(jnp.int32)

    info = plsc.get_sparse_core_info()
    NW = info.num_cores * info.num_subcores
    b_per_w = B // NW
    CH = 1600
    n_ch = b_per_w // CH
    mesh = plsc.VectorSubcoreMesh(core_axis_name="c", subcore_axis_name="s")

    @functools.partial(
        pl.kernel,
        mesh=mesh,
        out_type=jax.ShapeDtypeStruct((B, D), jnp.float32),
        scratch_types=[
            pltpu.VMEM((2, CH), jnp.int32),
            pltpu.VMEM((2, CH, D), jnp.float32),
            pltpu.SemaphoreType.DMA((2,)),
            pltpu.SemaphoreType.DMA((2,)),
            pltpu.SemaphoreType.DMA((2,)),
        ],
        compiler_params=pltpu.CompilerParams(use_tc_tiling_on_sc=False),
    )
    def emb(idx_hbm, table_hbm, out_hbm, idx_v, rows_v, sem_i, sem_g, sem_s):
        wid = lax.axis_index("s") * info.num_cores + lax.axis_index("c")
        base0 = wid * b_per_w

        def idx_start(j, b):
            pltpu.async_copy(
                idx_hbm.at[pl.ds(base0 + j * CH, CH)], idx_v.at[b], sem_i.at[b]
            )

        def idx_wait(b):
            pltpu.make_async_copy(
                idx_hbm.at[pl.ds(base0, CH)], idx_v.at[b], sem_i.at[b]
            ).wait()

        def gather_start(b):
            pltpu.async_copy(table_hbm.at[idx_v.at[b]], rows_v.at[b], sem_g.at[b])

        def gather_wait(b):
            pltpu.make_async_copy(
                table_hbm.at[idx_v.at[b]], rows_v.at[b], sem_g.at[b]
            ).wait()

        def store_start(j, b):
            pltpu.async_copy(
                rows_v.at[b], out_hbm.at[pl.ds(base0 + j * CH, CH)], sem_s.at[b]
            )

        def store_wait(b):
            pltpu.make_async_copy(
                rows_v.at[b], out_hbm.at[pl.ds(base0, CH)], sem_s.at[b]
            ).wait()

        idx_start(0, 0)
        idx_start(1, 1)
        idx_wait(0)
        gather_start(0)

        for j in range(n_ch):
            b = j % 2
            nb = 1 - b
            gather_wait(b)
            if j + 2 < n_ch:
                idx_start(j + 2, b)
            store_start(j, b)
            if j + 1 < n_ch:
                if j >= 1:
                    store_wait(nb)
                idx_wait(nb)
                gather_start(nb)
        store_wait((n_ch - 2) % 2)
        store_wait((n_ch - 1) % 2)

    out = emb(idx, table)
    return out.reshape(x.shape[0], x.shape[1], D)


# speedup vs baseline: 1.8053x; 1.6271x over previous
import functools

import jax
import jax.numpy as jnp
from jax import lax
from jax.experimental import pallas as pl
from jax.experimental.pallas import tpu as pltpu
from jax.experimental.pallas import tpu_sc as plsc


def kernel(x, table):
    BATCH, HIST = x.shape
    D = table.shape[1]

    info = plsc.get_sparse_core_info()
    NW = info.num_cores * info.num_subcores
    rows_w = BATCH // NW
    NB = 32
    n_ch = rows_w // NB
    mesh = plsc.VectorSubcoreMesh(core_axis_name="c", subcore_axis_name="s")

    @functools.partial(
        pl.kernel,
        mesh=mesh,
        out_type=jax.ShapeDtypeStruct((BATCH, HIST, D), jnp.float32),
        scratch_types=[
            pltpu.VMEM((rows_w, HIST), jnp.int32),
            pltpu.VMEM((2, NB, HIST, D), jnp.float32),
            pltpu.SemaphoreType.DMA,
            pltpu.SemaphoreType.DMA((2,)),
            pltpu.SemaphoreType.DMA((2,)),
        ],
        compiler_params=pltpu.CompilerParams(use_tc_tiling_on_sc=False),
    )
    def emb(x_hbm, table_hbm, out_hbm, idx_v, rows_v, sem_i, sem_g, sem_s):
        wid = lax.axis_index("s") * info.num_cores + lax.axis_index("c")
        row0 = wid * rows_w

        pltpu.async_copy(x_hbm.at[pl.ds(row0, rows_w), :], idx_v, sem_i).wait()

        def gather_start(j, b):
            def one(r, carry):
                pltpu.async_copy(
                    table_hbm.at[idx_v.at[j * NB + r]],
                    rows_v.at[b, r],
                    sem_g.at[b],
                )
                return carry

            lax.fori_loop(0, NB, one, 0)

        def gather_wait(b):
            pltpu.make_async_copy(
                out_hbm.at[pl.ds(0, NB)], rows_v.at[b], sem_g.at[b]
            ).wait()

        def store_start(j, b):
            pltpu.async_copy(
                rows_v.at[b], out_hbm.at[pl.ds(row0 + j * NB, NB)], sem_s.at[b]
            )

        def store_wait(b):
            pltpu.make_async_copy(
                rows_v.at[b], out_hbm.at[pl.ds(row0, NB)], sem_s.at[b]
            ).wait()

        gather_start(0, 0)
        gather_wait(0)
        store_start(0, 0)
        gather_start(1, 1)

        def body(i, carry):
            j1 = 2 * i + 1
            gather_wait(1)
            store_start(j1, 1)
            store_wait(0)
            gather_start(j1 + 1, 0)
            gather_wait(0)
            store_start(j1 + 1, 0)
            store_wait(1)
            gather_start(j1 + 2, 1)
            return carry

        lax.fori_loop(0, (n_ch - 2) // 2, body, 0)

        gather_wait(1)
        store_start(n_ch - 1, 1)
        store_wait(0)
        store_wait(1)

    return emb(x.astype(jnp.int32), table)
